# baseline (speedup 1.0000x reference)
import functools

import jax
import jax.numpy as jnp
from jax import lax
from jax.experimental import pallas as pl
from jax.experimental.pallas import tpu as pltpu
from jax.experimental.pallas import tpu_sc as plsc

N = 10000
E = 320000
D_IN = 128
H = 64
C = 2

NC = 2
NS = 16
NW = NC * NS

NPAD = 10240
RPT = NPAD // NS
CHUNK = 96
NCHUNKS = 105
EPT = NCHUNKS * CHUNK
EPAD = NW * EPT
CW = 16


def _fill_rows(ref, ncols, value):
    def row(r, carry):
        for q in range(ncols // 16):
            ref[r, pl.ds(q * 16, 16)] = jnp.full((16,), value, jnp.float32)
        return carry

    lax.fori_loop(0, CHUNK, row, 0)


def _zero_acc_slice(buf, acc, r0):
    for q in range(RPT // CHUNK):
        pltpu.sync_copy(buf, acc.at[pl.ds(r0 + q * CHUNK, CHUNK)])


def _sc_degree_body(dst_hbm, out_hbm, didx0, didx1,
                    ones_v, acc, sem_x0, sem_x1):
    c = lax.axis_index("c")
    s = lax.axis_index("s")
    wid = c * NS + s
    r0 = s * RPT
    base = wid * EPT
    _fill_rows(ones_v, CW, 0.0)
    _zero_acc_slice(ones_v, acc, r0)
    _fill_rows(ones_v, CW, 1.0)
    plsc.subcore_barrier()

    def idx_start(k, didx, sem_x):
        pltpu.async_copy(dst_hbm.at[pl.ds(base + k * CHUNK, CHUNK)], didx,
                         sem_x)

    def finish(k, didx, sem_x):
        pltpu.make_async_copy(dst_hbm.at[pl.ds(base + k * CHUNK, CHUNK)],
                              didx, sem_x).wait()
        pltpu.sync_copy(ones_v, acc.at[didx], add=True)

    def half(k, didx_c, sem_c, didx_n, sem_n):
        idx_start(k + 1, didx_n, sem_n)
        finish(k, didx_c, sem_c)

    idx_start(0, didx0, sem_x0)

    def body(j, carry):
        k = 2 * j
        half(k, didx0, sem_x0, didx1, sem_x1)
        half(k + 1, didx1, sem_x1, didx0, sem_x0)
        return carry

    lax.fori_loop(0, (NCHUNKS - 1) // 2, body, 0)
    finish(NCHUNKS - 1, didx0, sem_x0)

    plsc.subcore_barrier()
    pltpu.sync_copy(acc.at[pl.ds(r0, RPT)], out_hbm.at[pl.ds(c * NPAD + r0, RPT)])


def _sc_agg_body(src_hbm, dst_hbm, g_hbm, out_hbm,
                 sidx0, didx0, sidx1, didx1, rows0, rows1, acc,
                 sem_x0, sem_x1, sem_g0, sem_g1):
    c = lax.axis_index("c")
    s = lax.axis_index("s")
    wid = c * NS + s
    r0 = s * RPT
    base = wid * EPT
    _fill_rows(rows0, H, 0.0)
    _zero_acc_slice(rows0, acc, r0)
    plsc.subcore_barrier()

    p0 = (sidx0, didx0, rows0, sem_x0, sem_g0)
    p1 = (sidx1, didx1, rows1, sem_x1, sem_g1)

    def idx_start(k, p):
        sidx, didx, _, sem_x, _ = p
        off = base + k * CHUNK
        pltpu.async_copy(src_hbm.at[pl.ds(off, CHUNK)], sidx, sem_x)
        pltpu.async_copy(dst_hbm.at[pl.ds(off, CHUNK)], didx, sem_x)

    def idx_wait(k, p):
        sidx, didx, _, sem_x, _ = p
        off = base + k * CHUNK
        pltpu.make_async_copy(src_hbm.at[pl.ds(off, CHUNK)], sidx, sem_x).wait()
        pltpu.make_async_copy(dst_hbm.at[pl.ds(off, CHUNK)], didx, sem_x).wait()

    def gather_start(p):
        sidx, _, rows, _, sem_g = p
        pltpu.async_copy(g_hbm.at[sidx], rows, sem_g)

    def gather_wait(p):
        sidx, _, rows, _, sem_g = p
        pltpu.make_async_copy(g_hbm.at[sidx], rows, sem_g).wait()

    def scatter(p):
        _, didx, rows, _, _ = p
        pltpu.sync_copy(rows, acc.at[didx], add=True)

    def half(k, p_cur, p_nxt):
        idx_wait(k + 1, p_nxt)
        gather_start(p_nxt)
        gather_wait(p_cur)
        scatter(p_cur)

        @pl.when(k + 2 < NCHUNKS)
        def _():
            idx_start(k + 2, p_cur)

    idx_start(0, p0)
    idx_start(1, p1)
    idx_wait(0, p0)
    gather_start(p0)

    def body(j, carry):
        k = 2 * j
        half(k, p0, p1)
        half(k + 1, p1, p0)
        return carry

    lax.fori_loop(0, (NCHUNKS - 1) // 2, body, 0)
    gather_wait(p0)
    scatter(p0)

    plsc.subcore_barrier()
    pltpu.sync_copy(acc.at[pl.ds(r0, RPT)], out_hbm.at[pl.ds(c * NPAD + r0, RPT)])


@functools.lru_cache(maxsize=None)
def _sc_kernels():
    mesh = plsc.VectorSubcoreMesh(core_axis_name="c", subcore_axis_name="s")
    params = pltpu.CompilerParams(use_tc_tiling_on_sc=False)
    deg = pl.kernel(
        _sc_degree_body,
        mesh=mesh,
        out_type=jax.ShapeDtypeStruct((NC * NPAD, CW), jnp.float32),
        scratch_types=[
            pltpu.VMEM((CHUNK,), jnp.int32),
            pltpu.VMEM((CHUNK,), jnp.int32),
            pltpu.VMEM((CHUNK, CW), jnp.float32),
            pltpu.VMEM_SHARED((NPAD, CW), jnp.float32),
            pltpu.SemaphoreType.DMA,
            pltpu.SemaphoreType.DMA,
        ],
        compiler_params=params,
    )
    agg = pl.kernel(
        _sc_agg_body,
        mesh=mesh,
        out_type=jax.ShapeDtypeStruct((NC * NPAD, H), jnp.float32),
        scratch_types=[
            pltpu.VMEM((CHUNK,), jnp.int32),
            pltpu.VMEM((CHUNK,), jnp.int32),
            pltpu.VMEM((CHUNK,), jnp.int32),
            pltpu.VMEM((CHUNK,), jnp.int32),
            pltpu.VMEM((CHUNK, H), jnp.float32),
            pltpu.VMEM((CHUNK, H), jnp.float32),
            pltpu.VMEM_SHARED((NPAD, H), jnp.float32),
            pltpu.SemaphoreType.DMA,
            pltpu.SemaphoreType.DMA,
            pltpu.SemaphoreType.DMA,
            pltpu.SemaphoreType.DMA,
        ],
        compiler_params=params,
    )
    return deg, agg


_RB = 2000


def _dis_from_cnt(cnt):
    deg = cnt[0, :, :1] + cnt[1, :, :1] + 1.0
    return lax.rsqrt(deg)


def _k1_body(x_ref, w_ref, cnt_ref, g_ref):
    dis = _dis_from_cnt(cnt_ref[...])
    h = jnp.dot(x_ref[...], w_ref[...], preferred_element_type=jnp.float32)
    g_ref[...] = dis * h


def _kmid_body(scat_ref, g_ref, cnt_ref, w_ref, b_ref, gout_ref):
    dis = _dis_from_cnt(cnt_ref[...])
    sc = scat_ref[0] + scat_ref[1] + g_ref[...]
    act = jnp.maximum(dis * sc + b_ref[...], 0.0)
    gout_ref[...] = dis * jnp.dot(act, w_ref[...],
                                  preferred_element_type=jnp.float32)


def _kfin_body(scat_ref, g_ref, cnt_ref, b_ref, wc1_ref, bc1_ref, wc2_ref,
               bc2_ref, out_ref):
    cnt = cnt_ref[...]
    deg = cnt[0, :N, :1] + cnt[1, :N, :1] + 1.0
    dis = lax.rsqrt(deg)
    sc = scat_ref[0, :N] + scat_ref[1, :N] + g_ref[...]
    act = jnp.maximum(dis * sc + b_ref[...], 0.0)
    r = jnp.sum(act, axis=0, keepdims=True) * (1.0 / N)
    comb = jnp.concatenate([r, r], axis=1)
    z = jnp.maximum(
        jnp.dot(comb, wc1_ref[...], preferred_element_type=jnp.float32)
        + bc1_ref[...], 0.0)
    out_ref[...] = (jnp.dot(z, wc2_ref[...], preferred_element_type=jnp.float32)
                    + bc2_ref[...])


def _tc_layer1(x, W1, cnt):
    return pl.pallas_call(
        _k1_body,
        grid=(N // _RB,),
        in_specs=[
            pl.BlockSpec((_RB, D_IN), lambda i: (i, 0)),
            pl.BlockSpec((D_IN, H), lambda i: (0, 0)),
            pl.BlockSpec((2, _RB, CW), lambda i: (0, i, 0)),
        ],
        out_specs=pl.BlockSpec((_RB, H), lambda i: (i, 0)),
        out_shape=jax.ShapeDtypeStruct((N, H), jnp.float32),
    )(x, W1, cnt)


def _tc_mid(scat, g, cnt, W, b):
    return pl.pallas_call(
        _kmid_body,
        grid=(N // _RB,),
        in_specs=[
            pl.BlockSpec((2, _RB, H), lambda i: (0, i, 0)),
            pl.BlockSpec((_RB, H), lambda i: (i, 0)),
            pl.BlockSpec((2, _RB, CW), lambda i: (0, i, 0)),
            pl.BlockSpec((H, H), lambda i: (0, 0)),
            pl.BlockSpec((1, H), lambda i: (0, 0)),
        ],
        out_specs=pl.BlockSpec((_RB, H), lambda i: (i, 0)),
        out_shape=jax.ShapeDtypeStruct((N, H), jnp.float32),
    )(scat, g, cnt, W, b)


def _tc_final(scat, g, cnt, b3, Wc1, bc1, Wc2, bc2):
    return pl.pallas_call(
        _kfin_body,
        out_shape=jax.ShapeDtypeStruct((1, C), jnp.float32),
    )(scat, g, cnt, b3, Wc1, bc1, Wc2, bc2)


def kernel(x, edge_index, W1, b1, W2, b2, W3, b3, Wc1, bc1, Wc2, bc2):
    if EPAD > E:
        src = jnp.concatenate([edge_index[0],
                               jnp.zeros((EPAD - E,), jnp.int32)])
        pad_dst = N + jnp.arange(EPAD - E, dtype=jnp.int32) % (NPAD - N)
        dst = jnp.concatenate([edge_index[1], pad_dst])
    else:
        src = edge_index[0]
        dst = edge_index[1]

    sc_degree, sc_agg = _sc_kernels()
    cnt = sc_degree(dst).reshape(NC, NPAD, CW)

    g1 = _tc_layer1(x, W1, cnt)
    s1 = sc_agg(src, dst, g1).reshape(NC, NPAD, H)
    g2 = _tc_mid(s1, g1, cnt, W2, b1.reshape(1, H))
    s2 = sc_agg(src, dst, g2).reshape(NC, NPAD, H)
    g3 = _tc_mid(s2, g2, cnt, W3, b2.reshape(1, H))
    s3 = sc_agg(src, dst, g3).reshape(NC, NPAD, H)
    return _tc_final(s3, g3, cnt, b3.reshape(1, H), Wc1, bc1.reshape(1, H),
                     Wc2, bc2.reshape(1, C))


# speedup vs baseline: 1.5904x; 1.5904x over previous
import functools

import jax
import jax.numpy as jnp
from jax import lax
from jax.experimental import pallas as pl
from jax.experimental.pallas import tpu as pltpu
from jax.experimental.pallas import tpu_sc as plsc

N = 10000
E = 320000
D_IN = 128
H = 64
C = 2

NC = 2
NS = 16
NW = NC * NS

NPAD = 10240
RPT = NPAD // NS
CHUNK = 80
NCHUNKS = 125
EPT = NCHUNKS * CHUNK
EPAD = NW * EPT
CW = 16


def _fill_rows(ref, ncols, value):
    def row(r, carry):
        for q in range(ncols // 16):
            ref[r, pl.ds(q * 16, 16)] = jnp.full((16,), value, jnp.float32)
        return carry

    lax.fori_loop(0, CHUNK, row, 0)


def _zero_acc_slice(buf, acc, r0):
    for q in range(RPT // 64):
        pltpu.sync_copy(buf.at[pl.ds(0, 64)], acc.at[pl.ds(r0 + q * 64, 64)])


def _sc_degree_body(dst_hbm, out_hbm, didx0, didx1,
                    ones_v, acc, sem_x0, sem_x1):
    c = lax.axis_index("c")
    s = lax.axis_index("s")
    wid = c * NS + s
    r0 = s * RPT
    base = wid * EPT
    _fill_rows(ones_v, CW, 0.0)
    _zero_acc_slice(ones_v, acc, r0)
    _fill_rows(ones_v, CW, 1.0)
    plsc.subcore_barrier()

    def idx_start(k, didx, sem_x):
        pltpu.async_copy(dst_hbm.at[pl.ds(base + k * CHUNK, CHUNK)], didx,
                         sem_x)

    def finish(k, didx, sem_x):
        pltpu.make_async_copy(dst_hbm.at[pl.ds(base + k * CHUNK, CHUNK)],
                              didx, sem_x).wait()
        pltpu.sync_copy(ones_v, acc.at[didx], add=True)

    def half(k, didx_c, sem_c, didx_n, sem_n):
        idx_start(k + 1, didx_n, sem_n)
        finish(k, didx_c, sem_c)

    idx_start(0, didx0, sem_x0)

    def body(j, carry):
        k = 2 * j
        half(k, didx0, sem_x0, didx1, sem_x1)
        half(k + 1, didx1, sem_x1, didx0, sem_x0)
        return carry

    lax.fori_loop(0, (NCHUNKS - 1) // 2, body, 0)
    finish(NCHUNKS - 1, didx0, sem_x0)

    plsc.subcore_barrier()
    pltpu.sync_copy(acc.at[pl.ds(r0, RPT)], out_hbm.at[pl.ds(c * NPAD + r0, RPT)])


def _sc_agg_body(src_hbm, dst_hbm, g_hbm, out_hbm,
                 sidx0, sidx1, sidx2, sidx3,
                 didx0, didx1, didx2, didx3,
                 rows0, rows1, rows2, rows3, acc,
                 sx0, sx1, sx2, sx3, sg0, sg1, sg2, sg3, ss0, ss1, ss2, ss3):
    c = lax.axis_index("c")
    s = lax.axis_index("s")
    wid = c * NS + s
    r0 = s * RPT
    base = wid * EPT
    _fill_rows(rows0, H, 0.0)
    _zero_acc_slice(rows0, acc, r0)
    plsc.subcore_barrier()

    slots = [
        (sidx0, didx0, rows0, sx0, sg0, ss0),
        (sidx1, didx1, rows1, sx1, sg1, ss1),
        (sidx2, didx2, rows2, sx2, sg2, ss2),
        (sidx3, didx3, rows3, sx3, sg3, ss3),
    ]

    def idx_start(k, p):
        sidx, didx, _, sem_x, _, _ = p
        off = base + k * CHUNK
        pltpu.async_copy(src_hbm.at[pl.ds(off, CHUNK)], sidx, sem_x)
        pltpu.async_copy(dst_hbm.at[pl.ds(off, CHUNK)], didx, sem_x)

    def idx_wait(k, p):
        sidx, didx, _, sem_x, _, _ = p
        off = base + k * CHUNK
        pltpu.make_async_copy(src_hbm.at[pl.ds(off, CHUNK)], sidx, sem_x).wait()
        pltpu.make_async_copy(dst_hbm.at[pl.ds(off, CHUNK)], didx, sem_x).wait()

    def gather_start(p):
        sidx, _, rows, _, sem_g, _ = p
        pltpu.async_copy(g_hbm.at[sidx], rows, sem_g)

    def gather_wait(p):
        sidx, _, rows, _, sem_g, _ = p
        pltpu.make_async_copy(g_hbm.at[sidx], rows, sem_g).wait()

    def scatter_start(p):
        _, didx, rows, _, _, sem_s = p
        pltpu.async_copy(rows, acc.at[didx], sem_s, add=True)

    def scatter_wait(p):
        _, didx, rows, _, _, sem_s = p
        pltpu.make_async_copy(rows, acc.at[didx], sem_s).wait()

    def slot(k, t):
        p_cur = slots[t]
        idx_wait(k + 1, slots[(t + 1) % 4])
        gather_start(slots[(t + 1) % 4])
        gather_wait(p_cur)

        @pl.when(k > 0)
        def _():
            scatter_wait(slots[(t + 3) % 4])

        scatter_start(p_cur)

        @pl.when(k + 3 < NCHUNKS)
        def _():
            idx_start(k + 3, slots[(t + 3) % 4])

    idx_start(0, slots[0])
    idx_start(1, slots[1])
    idx_start(2, slots[2])
    idx_wait(0, slots[0])
    gather_start(slots[0])

    def body(j, carry):
        k = 4 * j
        slot(k, 0)
        slot(k + 1, 1)
        slot(k + 2, 2)
        slot(k + 3, 3)
        return carry

    lax.fori_loop(0, (NCHUNKS - 1) // 4, body, 0)
    gather_wait(slots[0])
    scatter_wait(slots[3])
    scatter_start(slots[0])
    scatter_wait(slots[0])

    plsc.subcore_barrier()
    pltpu.sync_copy(acc.at[pl.ds(r0, RPT)], out_hbm.at[pl.ds(c * NPAD + r0, RPT)])


@functools.lru_cache(maxsize=None)
def _sc_kernels():
    mesh = plsc.VectorSubcoreMesh(core_axis_name="c", subcore_axis_name="s")
    params = pltpu.CompilerParams(use_tc_tiling_on_sc=False)
    deg = pl.kernel(
        _sc_degree_body,
        mesh=mesh,
        out_type=jax.ShapeDtypeStruct((NC * NPAD, CW), jnp.float32),
        scratch_types=[
            pltpu.VMEM((CHUNK,), jnp.int32),
            pltpu.VMEM((CHUNK,), jnp.int32),
            pltpu.VMEM((CHUNK, CW), jnp.float32),
            pltpu.VMEM_SHARED((NPAD, CW), jnp.float32),
            pltpu.SemaphoreType.DMA,
            pltpu.SemaphoreType.DMA,
        ],
        compiler_params=params,
    )
    agg = pl.kernel(
        _sc_agg_body,
        mesh=mesh,
        out_type=jax.ShapeDtypeStruct((NC * NPAD, H), jnp.float32),
        scratch_types=(
            [pltpu.VMEM((CHUNK,), jnp.int32) for _ in range(8)]
            + [pltpu.VMEM((CHUNK, H), jnp.float32) for _ in range(4)]
            + [pltpu.VMEM_SHARED((NPAD, H), jnp.float32)]
            + [pltpu.SemaphoreType.DMA for _ in range(12)]
        ),
        compiler_params=params,
    )
    return deg, agg


_RB = 2000


def _dis_from_cnt(cnt):
    deg = cnt[0, :, :1] + cnt[1, :, :1] + 1.0
    return lax.rsqrt(deg)


def _k1_body(x_ref, w_ref, cnt_ref, g_ref):
    dis = _dis_from_cnt(cnt_ref[...])
    h = jnp.dot(x_ref[...], w_ref[...], preferred_element_type=jnp.float32)
    g_ref[...] = dis * h


def _kmid_body(scat_ref, g_ref, cnt_ref, w_ref, b_ref, gout_ref):
    dis = _dis_from_cnt(cnt_ref[...])
    sc = scat_ref[0] + scat_ref[1] + g_ref[...]
    act = jnp.maximum(dis * sc + b_ref[...], 0.0)
    gout_ref[...] = dis * jnp.dot(act, w_ref[...],
                                  preferred_element_type=jnp.float32)


def _kfin_body(scat_ref, g_ref, cnt_ref, b_ref, wc1_ref, bc1_ref, wc2_ref,
               bc2_ref, out_ref):
    cnt = cnt_ref[...]
    deg = cnt[0, :N, :1] + cnt[1, :N, :1] + 1.0
    dis = lax.rsqrt(deg)
    sc = scat_ref[0, :N] + scat_ref[1, :N] + g_ref[...]
    act = jnp.maximum(dis * sc + b_ref[...], 0.0)
    r = jnp.sum(act, axis=0, keepdims=True) * (1.0 / N)
    comb = jnp.concatenate([r, r], axis=1)
    z = jnp.maximum(
        jnp.dot(comb, wc1_ref[...], preferred_element_type=jnp.float32)
        + bc1_ref[...], 0.0)
    out_ref[...] = (jnp.dot(z, wc2_ref[...], preferred_element_type=jnp.float32)
                    + bc2_ref[...])


def _tc_layer1(x, W1, cnt):
    return pl.pallas_call(
        _k1_body,
        grid=(N // _RB,),
        in_specs=[
            pl.BlockSpec((_RB, D_IN), lambda i: (i, 0)),
            pl.BlockSpec((D_IN, H), lambda i: (0, 0)),
            pl.BlockSpec((2, _RB, CW), lambda i: (0, i, 0)),
        ],
        out_specs=pl.BlockSpec((_RB, H), lambda i: (i, 0)),
        out_shape=jax.ShapeDtypeStruct((N, H), jnp.float32),
    )(x, W1, cnt)


def _tc_mid(scat, g, cnt, W, b):
    return pl.pallas_call(
        _kmid_body,
        grid=(N // _RB,),
        in_specs=[
            pl.BlockSpec((2, _RB, H), lambda i: (0, i, 0)),
            pl.BlockSpec((_RB, H), lambda i: (i, 0)),
            pl.BlockSpec((2, _RB, CW), lambda i: (0, i, 0)),
            pl.BlockSpec((H, H), lambda i: (0, 0)),
            pl.BlockSpec((1, H), lambda i: (0, 0)),
        ],
        out_specs=pl.BlockSpec((_RB, H), lambda i: (i, 0)),
        out_shape=jax.ShapeDtypeStruct((N, H), jnp.float32),
    )(scat, g, cnt, W, b)


def _tc_final(scat, g, cnt, b3, Wc1, bc1, Wc2, bc2):
    return pl.pallas_call(
        _kfin_body,
        out_shape=jax.ShapeDtypeStruct((1, C), jnp.float32),
    )(scat, g, cnt, b3, Wc1, bc1, Wc2, bc2)


def kernel(x, edge_index, W1, b1, W2, b2, W3, b3, Wc1, bc1, Wc2, bc2):
    if EPAD > E:
        src = jnp.concatenate([edge_index[0],
                               jnp.zeros((EPAD - E,), jnp.int32)])
        pad_dst = N + jnp.arange(EPAD - E, dtype=jnp.int32) % (NPAD - N)
        dst = jnp.concatenate([edge_index[1], pad_dst])
    else:
        src = edge_index[0]
        dst = edge_index[1]

    sc_degree, sc_agg = _sc_kernels()
    cnt = sc_degree(dst).reshape(NC, NPAD, CW)

    g1 = _tc_layer1(x, W1, cnt)
    s1 = sc_agg(src, dst, g1).reshape(NC, NPAD, H)
    g2 = _tc_mid(s1, g1, cnt, W2, b1.reshape(1, H))
    s2 = sc_agg(src, dst, g2).reshape(NC, NPAD, H)
    g3 = _tc_mid(s2, g2, cnt, W3, b2.reshape(1, H))
    s3 = sc_agg(src, dst, g3).reshape(NC, NPAD, H)
    return _tc_final(s3, g3, cnt, b3.reshape(1, H), Wc1, bc1.reshape(1, H),
                     Wc2, bc2.reshape(1, C))
